# Initial kernel scaffold; baseline (speedup 1.0000x reference)
import jax
import jax.numpy as jnp
from jax.experimental import pallas as pl


def kernel(edge_index, features, W_gcn, b_gcn, W_t, b_t):
    raise NotImplementedError("write your pallas kernel here")


# speedup vs baseline: 10.8406x; 10.8406x over previous
import functools

import jax
import jax.numpy as jnp
import numpy as np
from jax import lax
from jax.experimental import pallas as pl
from jax.experimental.pallas import tpu as pltpu
from jax.experimental.pallas import tpu_sc as plsc

N = 10000
E = 320000
D = 128
K = 16

NC = 2
NS = 16
NW = NC * NS
EPT = E // NW
CH = 128
NFULL = EPT // CH
TAIL = EPT - NFULL * CH
WB_A = 624
WB_OFF = WB_A * (NS - 1)
WB_B = N - WB_OFF
ZCH = 104

SELU_ALPHA = 1.6732632423543772
SELU_SCALE = 1.0507009873554805


def _sc_mesh():
    return plsc.VectorSubcoreMesh(
        core_axis_name="c", subcore_axis_name="s", num_cores=NC, num_subcores=NS
    )


def _zero_my_slice(zbuf, table, s):
    zeros = jnp.zeros((16,), jnp.float32)

    def zb(i, carry):
        zbuf[i // 8, pl.ds((i % 8) * 16, 16)] = zeros
        return carry

    lax.fori_loop(0, ZCH * 8, zb, 0)
    base = s * WB_A
    for j in range(6):
        pltpu.sync_copy(zbuf, table.at[pl.ds(base + j * ZCH, ZCH)])

    @pl.when(s == NS - 1)
    def _():
        pltpu.sync_copy(zbuf.at[pl.ds(0, 16)], table.at[pl.ds(N - 16, 16)])


def _write_back(table, out_hbm, c, s):
    base = s * WB_A
    for j in range(2):
        pltpu.sync_copy(table.at[pl.ds(base + j * 312, 312)],
                        out_hbm.at[c, pl.ds(base + j * 312, 312)])

    @pl.when(s == NS - 1)
    def _():
        pltpu.sync_copy(table.at[pl.ds(N - 16, 16)],
                        out_hbm.at[c, pl.ds(N - 16, 16)])


@functools.partial(
    pl.kernel,
    out_type=jax.ShapeDtypeStruct((NC, N, D), jnp.float32),
    mesh=_sc_mesh(),
    scratch_types=[
        pltpu.VMEM((CH,), jnp.int32),
        pltpu.VMEM((CH,), jnp.int32),
        pltpu.VMEM((TAIL,), jnp.int32),
        pltpu.VMEM((TAIL,), jnp.int32),
        pltpu.VMEM((CH, D), jnp.float32),
        pltpu.VMEM((CH, D), jnp.float32),
        pltpu.VMEM((ZCH, D), jnp.float32),
        pltpu.VMEM_SHARED((N, D), jnp.float32),
    ],
)
def _deg_sc_kernel(srci_hbm, dsti_hbm, h_out,
                   sbuf, dbuf, sbuf_t, dbuf_t, e0, e1, zbuf, htab):
    c = lax.axis_index("c")
    s = lax.axis_index("s")
    wid = s * NC + c
    ones = jnp.ones((16,), jnp.float32)
    zeros = jnp.zeros((16,), jnp.float32)

    def fill(i, carry):
        r = i // 8
        col = (i % 8) * 16
        e0[r, pl.ds(col, 16)] = zeros
        e1[r, pl.ds(col, 16)] = zeros
        return carry

    lax.fori_loop(0, CH * 8, fill, 0)

    def mark(i, carry):
        e0[i, pl.ds(0, 16)] = ones
        e1[i, pl.ds(64, 16)] = ones
        return carry

    lax.fori_loop(0, CH, mark, 0)
    _zero_my_slice(zbuf, htab, s)
    plsc.subcore_barrier()

    base = wid * EPT

    def chunk(k, carry):
        off = base + k * CH
        pltpu.sync_copy(srci_hbm.at[pl.ds(off, CH)], sbuf)
        pltpu.sync_copy(dsti_hbm.at[pl.ds(off, CH)], dbuf)
        pltpu.sync_copy(e0, htab.at[sbuf], add=True)
        pltpu.sync_copy(e1, htab.at[dbuf], add=True)
        return carry

    lax.fori_loop(0, NFULL, chunk, 0)

    offt = base + NFULL * CH
    pltpu.sync_copy(srci_hbm.at[pl.ds(offt, TAIL)], sbuf_t)
    pltpu.sync_copy(dsti_hbm.at[pl.ds(offt, TAIL)], dbuf_t)
    pltpu.sync_copy(e0.at[pl.ds(0, TAIL)], htab.at[sbuf_t], add=True)
    pltpu.sync_copy(e1.at[pl.ds(0, TAIL)], htab.at[dbuf_t], add=True)

    plsc.subcore_barrier()
    _write_back(htab, h_out, c, s)


def _deg_body(h0_ref, h1_ref, rs_dr_ref, rs_dc_ref, dc_ref):
    dr = h0_ref[:, 0:1] + h1_ref[:, 0:1]
    dc = h0_ref[:, 64:65] + h1_ref[:, 64:65]
    rs_dr_ref[...] = lax.rsqrt(jnp.maximum(dr, 1.0))
    rs_dc_ref[...] = lax.rsqrt(jnp.maximum(dc, 1.0))
    dc_ref[...] = dc


_deg_call = pl.pallas_call(
    _deg_body,
    out_shape=(
        jax.ShapeDtypeStruct((N, 1), jnp.float32),
        jax.ShapeDtypeStruct((N, 1), jnp.float32),
        jax.ShapeDtypeStruct((N, 1), jnp.float32),
    ),
)


def _sup_body(x_ref, w_ref, b_ref, rs_ref, out_ref):
    acc = jnp.dot(x_ref[...], w_ref[...], preferred_element_type=jnp.float32)
    out_ref[...] = (acc + b_ref[...]) * rs_ref[...]


_sup_call = pl.pallas_call(
    _sup_body,
    out_shape=jax.ShapeDtypeStruct((N, D), jnp.float32),
)


@functools.partial(
    pl.kernel,
    out_type=jax.ShapeDtypeStruct((NC, N, D), jnp.float32),
    mesh=_sc_mesh(),
    scratch_types=[
        pltpu.VMEM((CH,), jnp.int32),
        pltpu.VMEM((CH,), jnp.int32),
        pltpu.VMEM((CH, D), jnp.float32),
        pltpu.VMEM((TAIL,), jnp.int32),
        pltpu.VMEM((TAIL,), jnp.int32),
        pltpu.VMEM((TAIL, D), jnp.float32),
        pltpu.VMEM((ZCH, D), jnp.float32),
        pltpu.VMEM_SHARED((N, D), jnp.float32),
        pltpu.SemaphoreType.DMA,
    ],
)
def _spmm_kernel(tab_hbm, dsti_hbm, srci_hbm, z_out,
                 dbuf, sbuf, rows, dbuf_t, sbuf_t, rows_t, zbuf, zacc, sem):
    c = lax.axis_index("c")
    s = lax.axis_index("s")
    wid = s * NC + c
    _zero_my_slice(zbuf, zacc, s)
    plsc.subcore_barrier()

    base = wid * EPT

    def chunk(k, carry):
        off = base + k * CH
        pltpu.sync_copy(dsti_hbm.at[pl.ds(off, CH)], dbuf)
        pltpu.sync_copy(srci_hbm.at[pl.ds(off, CH)], sbuf)
        pltpu.async_copy(tab_hbm.at[dbuf], rows, sem).wait()
        pltpu.sync_copy(rows, zacc.at[sbuf], add=True)
        return carry

    lax.fori_loop(0, NFULL, chunk, 0)

    offt = base + NFULL * CH
    pltpu.sync_copy(dsti_hbm.at[pl.ds(offt, TAIL)], dbuf_t)
    pltpu.sync_copy(srci_hbm.at[pl.ds(offt, TAIL)], sbuf_t)
    pltpu.async_copy(tab_hbm.at[dbuf_t], rows_t, sem).wait()
    pltpu.sync_copy(rows_t, zacc.at[sbuf_t], add=True)

    plsc.subcore_barrier()
    _write_back(zacc, z_out, c, s)


def _assign_body(z0_ref, z1_ref, rs_ref, wt_ref, bt_ref, dc_ref,
                 apad_ref, cs_ref, nl_ref):
    z = (z0_ref[...] + z1_ref[...]) * rs_ref[...]
    g = SELU_SCALE * jnp.where(z > 0, z, SELU_ALPHA * (jnp.exp(z) - 1.0))
    logits = jnp.dot(g, wt_ref[...], preferred_element_type=jnp.float32) + bt_ref[...]
    m = jnp.max(logits, axis=1, keepdims=True)
    e = jnp.exp(logits - m)
    a = e / jnp.sum(e, axis=1, keepdims=True)
    apad_ref[...] = jnp.concatenate(
        [a, jnp.zeros((N, D - K), jnp.float32)], axis=1
    )
    cs_ref[...] = jnp.sum(a, axis=0, keepdims=True)
    nl_ref[...] = jnp.sum(dc_ref[...] * a, axis=0, keepdims=True)


_assign_call = pl.pallas_call(
    _assign_body,
    out_shape=(
        jax.ShapeDtypeStruct((N, D), jnp.float32),
        jax.ShapeDtypeStruct((1, K), jnp.float32),
        jax.ShapeDtypeStruct((1, K), jnp.float32),
    ),
)


def _loss_body(as0_ref, as1_ref, apad_ref, cs_ref, nl_ref, loss_ref):
    tp = jnp.sum((as0_ref[...] + as1_ref[...]) * apad_ref[...])
    nl = nl_ref[...]
    cs = cs_ref[...]
    e_f = jnp.float32(E)
    tn = jnp.sum(nl * nl) / 2.0 / e_f
    spectral = -(tp - tn) / 2.0 / e_f
    cluster = jnp.sqrt(jnp.sum(cs * cs)) / jnp.float32(N) * np.sqrt(float(K)) - 1.0
    loss_ref[...] = jnp.full((1, 1), spectral + cluster, jnp.float32)


_loss_call = pl.pallas_call(
    _loss_body,
    out_shape=jax.ShapeDtypeStruct((1, 1), jnp.float32),
)


def kernel(edge_index, features, W_gcn, b_gcn, W_t, b_t):
    src = edge_index[0].astype(jnp.int32)
    dst = edge_index[1].astype(jnp.int32)

    h = _deg_sc_kernel(src, dst)
    rs_dr, rs_dc, dc_col = _deg_call(h[0], h[1])

    support2 = _sup_call(features, W_gcn, b_gcn.reshape(1, D), rs_dc)
    zp = _spmm_kernel(support2, dst, src)
    apad, cs, nl = _assign_call(
        zp[0], zp[1], rs_dr, W_t, b_t.reshape(1, K), dc_col
    )
    asp = _spmm_kernel(apad, dst, src)
    loss = _loss_call(asp[0], asp[1], apad, cs, nl)
    return loss[0, 0]


# speedup vs baseline: 14.3556x; 1.3242x over previous
import functools

import jax
import jax.numpy as jnp
import numpy as np
from jax import lax
from jax.experimental import pallas as pl
from jax.experimental.pallas import tpu as pltpu
from jax.experimental.pallas import tpu_sc as plsc

N = 10000
E = 320000
D = 128
K = 16

NC = 2
NS = 16
NW = NC * NS
EPT = E // NW
CH = 128
NFULL = EPT // CH
TAIL = EPT - NFULL * CH
WB_A = 624
WB_OFF = WB_A * (NS - 1)
WB_B = N - WB_OFF
ZCH = 104

SELU_ALPHA = 1.6732632423543772
SELU_SCALE = 1.0507009873554805


def _sc_mesh():
    return plsc.VectorSubcoreMesh(
        core_axis_name="c", subcore_axis_name="s", num_cores=NC, num_subcores=NS
    )


def _zero_my_slice(zbuf, table, s):
    zeros = jnp.zeros((16,), jnp.float32)

    def zb(i, carry):
        zbuf[i // 8, pl.ds((i % 8) * 16, 16)] = zeros
        return carry

    lax.fori_loop(0, ZCH * 8, zb, 0)
    base = s * WB_A
    for j in range(6):
        pltpu.sync_copy(zbuf, table.at[pl.ds(base + j * ZCH, ZCH)])

    @pl.when(s == NS - 1)
    def _():
        pltpu.sync_copy(zbuf.at[pl.ds(0, 16)], table.at[pl.ds(N - 16, 16)])


def _write_back(table, out_hbm, c, s):
    base = s * WB_A
    for j in range(2):
        pltpu.sync_copy(table.at[pl.ds(base + j * 312, 312)],
                        out_hbm.at[c, pl.ds(base + j * 312, 312)])

    @pl.when(s == NS - 1)
    def _():
        pltpu.sync_copy(table.at[pl.ds(N - 16, 16)],
                        out_hbm.at[c, pl.ds(N - 16, 16)])


@functools.partial(
    pl.kernel,
    out_type=jax.ShapeDtypeStruct((NC, N, D), jnp.float32),
    mesh=_sc_mesh(),
    scratch_types=[
        pltpu.VMEM((CH,), jnp.int32),
        pltpu.VMEM((CH,), jnp.int32),
        pltpu.VMEM((TAIL,), jnp.int32),
        pltpu.VMEM((TAIL,), jnp.int32),
        pltpu.VMEM((CH, D), jnp.float32),
        pltpu.VMEM((CH, D), jnp.float32),
        pltpu.VMEM((ZCH, D), jnp.float32),
        pltpu.VMEM_SHARED((N, D), jnp.float32),
    ],
)
def _deg_sc_kernel(srci_hbm, dsti_hbm, h_out,
                   sbuf, dbuf, sbuf_t, dbuf_t, e0, e1, zbuf, htab):
    c = lax.axis_index("c")
    s = lax.axis_index("s")
    wid = s * NC + c
    ones = jnp.ones((16,), jnp.float32)
    zeros = jnp.zeros((16,), jnp.float32)

    def fill(i, carry):
        r = i // 8
        col = (i % 8) * 16
        e0[r, pl.ds(col, 16)] = zeros
        e1[r, pl.ds(col, 16)] = zeros
        return carry

    lax.fori_loop(0, CH * 8, fill, 0)

    def mark(i, carry):
        e0[i, pl.ds(0, 16)] = ones
        e1[i, pl.ds(64, 16)] = ones
        return carry

    lax.fori_loop(0, CH, mark, 0)
    _zero_my_slice(zbuf, htab, s)
    plsc.subcore_barrier()

    base = wid * EPT

    def chunk(k, carry):
        off = base + k * CH
        pltpu.sync_copy(srci_hbm.at[pl.ds(off, CH)], sbuf)
        pltpu.sync_copy(dsti_hbm.at[pl.ds(off, CH)], dbuf)
        pltpu.sync_copy(e0, htab.at[sbuf], add=True)
        pltpu.sync_copy(e1, htab.at[dbuf], add=True)
        return carry

    lax.fori_loop(0, NFULL, chunk, 0)

    offt = base + NFULL * CH
    pltpu.sync_copy(srci_hbm.at[pl.ds(offt, TAIL)], sbuf_t)
    pltpu.sync_copy(dsti_hbm.at[pl.ds(offt, TAIL)], dbuf_t)
    pltpu.sync_copy(e0.at[pl.ds(0, TAIL)], htab.at[sbuf_t], add=True)
    pltpu.sync_copy(e1.at[pl.ds(0, TAIL)], htab.at[dbuf_t], add=True)

    plsc.subcore_barrier()
    _write_back(htab, h_out, c, s)


def _deg_body(h0_ref, h1_ref, rs_dr_ref, rs_dc_ref, dc_ref):
    dr = h0_ref[:, 0:1] + h1_ref[:, 0:1]
    dc = h0_ref[:, 64:65] + h1_ref[:, 64:65]
    rs_dr_ref[...] = lax.rsqrt(jnp.maximum(dr, 1.0))
    rs_dc_ref[...] = lax.rsqrt(jnp.maximum(dc, 1.0))
    dc_ref[...] = dc


_deg_call = pl.pallas_call(
    _deg_body,
    out_shape=(
        jax.ShapeDtypeStruct((N, 1), jnp.float32),
        jax.ShapeDtypeStruct((N, 1), jnp.float32),
        jax.ShapeDtypeStruct((N, 1), jnp.float32),
    ),
)


def _sup_body(x_ref, w_ref, b_ref, rs_ref, out_ref):
    acc = jnp.dot(x_ref[...], w_ref[...], preferred_element_type=jnp.float32)
    out_ref[...] = (acc + b_ref[...]) * rs_ref[...]


_sup_call = pl.pallas_call(
    _sup_body,
    out_shape=jax.ShapeDtypeStruct((N, D), jnp.float32),
)


@functools.partial(
    pl.kernel,
    out_type=jax.ShapeDtypeStruct((NC, N, D), jnp.float32),
    mesh=_sc_mesh(),
    scratch_types=[
        pltpu.VMEM((CH,), jnp.int32),
        pltpu.VMEM((CH,), jnp.int32),
        pltpu.VMEM((CH, D), jnp.float32),
        pltpu.VMEM((CH,), jnp.int32),
        pltpu.VMEM((CH,), jnp.int32),
        pltpu.VMEM((CH, D), jnp.float32),
        pltpu.VMEM((TAIL,), jnp.int32),
        pltpu.VMEM((TAIL,), jnp.int32),
        pltpu.VMEM((TAIL, D), jnp.float32),
        pltpu.VMEM((ZCH, D), jnp.float32),
        pltpu.VMEM_SHARED((N, D), jnp.float32),
        pltpu.SemaphoreType.DMA,
        pltpu.SemaphoreType.DMA,
    ],
)
def _spmm_kernel(tab_hbm, dsti_hbm, srci_hbm, z_out,
                 dbuf0, sbuf0, rows0, dbuf1, sbuf1, rows1,
                 dbuf_t, sbuf_t, rows_t, zbuf, zacc, sem0, sem1):
    c = lax.axis_index("c")
    s = lax.axis_index("s")
    wid = s * NC + c
    _zero_my_slice(zbuf, zacc, s)
    plsc.subcore_barrier()

    base = wid * EPT

    def load_idx(k, dbuf, sbuf):
        off = base + k * CH
        pltpu.sync_copy(dsti_hbm.at[pl.ds(off, CH)], dbuf)
        pltpu.sync_copy(srci_hbm.at[pl.ds(off, CH)], sbuf)

    load_idx(0, dbuf0, sbuf0)
    pltpu.async_copy(tab_hbm.at[dbuf0], rows0, sem0)

    def body(i, carry):
        load_idx(2 * i + 1, dbuf1, sbuf1)
        pltpu.async_copy(tab_hbm.at[dbuf1], rows1, sem1)
        pltpu.make_async_copy(tab_hbm.at[dbuf0], rows0, sem0).wait()
        pltpu.sync_copy(rows0, zacc.at[sbuf0], add=True)

        @pl.when(i < NFULL // 2 - 1)
        def _():
            load_idx(2 * i + 2, dbuf0, sbuf0)
            pltpu.async_copy(tab_hbm.at[dbuf0], rows0, sem0)

        pltpu.make_async_copy(tab_hbm.at[dbuf1], rows1, sem1).wait()
        pltpu.sync_copy(rows1, zacc.at[sbuf1], add=True)
        return carry

    lax.fori_loop(0, NFULL // 2, body, 0)

    offt = base + NFULL * CH
    pltpu.sync_copy(dsti_hbm.at[pl.ds(offt, TAIL)], dbuf_t)
    pltpu.sync_copy(srci_hbm.at[pl.ds(offt, TAIL)], sbuf_t)
    pltpu.async_copy(tab_hbm.at[dbuf_t], rows_t, sem0).wait()
    pltpu.sync_copy(rows_t, zacc.at[sbuf_t], add=True)

    plsc.subcore_barrier()
    _write_back(zacc, z_out, c, s)


def _assign_body(z0_ref, z1_ref, rs_ref, wt_ref, bt_ref, dc_ref,
                 apad_ref, cs_ref, nl_ref):
    z = (z0_ref[...] + z1_ref[...]) * rs_ref[...]
    g = SELU_SCALE * jnp.where(z > 0, z, SELU_ALPHA * (jnp.exp(z) - 1.0))
    logits = jnp.dot(g, wt_ref[...], preferred_element_type=jnp.float32) + bt_ref[...]
    m = jnp.max(logits, axis=1, keepdims=True)
    e = jnp.exp(logits - m)
    a = e / jnp.sum(e, axis=1, keepdims=True)
    apad_ref[...] = jnp.concatenate(
        [a, jnp.zeros((N, D - K), jnp.float32)], axis=1
    )
    cs_ref[...] = jnp.sum(a, axis=0, keepdims=True)
    nl_ref[...] = jnp.sum(dc_ref[...] * a, axis=0, keepdims=True)


_assign_call = pl.pallas_call(
    _assign_body,
    out_shape=(
        jax.ShapeDtypeStruct((N, D), jnp.float32),
        jax.ShapeDtypeStruct((1, K), jnp.float32),
        jax.ShapeDtypeStruct((1, K), jnp.float32),
    ),
)


def _loss_body(as0_ref, as1_ref, apad_ref, cs_ref, nl_ref, loss_ref):
    tp = jnp.sum((as0_ref[...] + as1_ref[...]) * apad_ref[...])
    nl = nl_ref[...]
    cs = cs_ref[...]
    e_f = jnp.float32(E)
    tn = jnp.sum(nl * nl) / 2.0 / e_f
    spectral = -(tp - tn) / 2.0 / e_f
    cluster = jnp.sqrt(jnp.sum(cs * cs)) / jnp.float32(N) * np.sqrt(float(K)) - 1.0
    loss_ref[...] = jnp.full((1, 1), spectral + cluster, jnp.float32)


_loss_call = pl.pallas_call(
    _loss_body,
    out_shape=jax.ShapeDtypeStruct((1, 1), jnp.float32),
)


def kernel(edge_index, features, W_gcn, b_gcn, W_t, b_t):
    src = edge_index[0].astype(jnp.int32)
    dst = edge_index[1].astype(jnp.int32)

    h = _deg_sc_kernel(src, dst)
    rs_dr, rs_dc, dc_col = _deg_call(h[0], h[1])

    support2 = _sup_call(features, W_gcn, b_gcn.reshape(1, D), rs_dc)
    zp = _spmm_kernel(support2, dst, src)
    apad, cs, nl = _assign_call(
        zp[0], zp[1], rs_dr, W_t, b_t.reshape(1, K), dc_col
    )
    asp = _spmm_kernel(apad, dst, src)
    loss = _loss_call(asp[0], asp[1], apad, cs, nl)
    return loss[0, 0]


# speedup vs baseline: 14.6319x; 1.0192x over previous
import functools

import jax
import jax.numpy as jnp
import numpy as np
from jax import lax
from jax.experimental import pallas as pl
from jax.experimental.pallas import tpu as pltpu
from jax.experimental.pallas import tpu_sc as plsc

N = 10000
E = 320000
D = 128
K = 16

NC = 2
NS = 16
NW = NC * NS
EPT = E // NW
CH = 128
NFULL = EPT // CH
TAIL = EPT - NFULL * CH
WB_A = 624
WB_OFF = WB_A * (NS - 1)
WB_B = N - WB_OFF
ZCH = 104

SELU_ALPHA = 1.6732632423543772
SELU_SCALE = 1.0507009873554805


def _sc_mesh():
    return plsc.VectorSubcoreMesh(
        core_axis_name="c", subcore_axis_name="s", num_cores=NC, num_subcores=NS
    )


def _zero_my_slice(zbuf, table, s):
    zeros = jnp.zeros((16,), jnp.float32)

    def zb(i, carry):
        zbuf[i // 8, pl.ds((i % 8) * 16, 16)] = zeros
        return carry

    lax.fori_loop(0, ZCH * 8, zb, 0)
    base = s * WB_A
    for j in range(6):
        pltpu.sync_copy(zbuf, table.at[pl.ds(base + j * ZCH, ZCH)])

    @pl.when(s == NS - 1)
    def _():
        pltpu.sync_copy(zbuf.at[pl.ds(0, 16)], table.at[pl.ds(N - 16, 16)])


def _write_back(table, out_hbm, c, s):
    base = s * WB_A
    for j in range(2):
        pltpu.sync_copy(table.at[pl.ds(base + j * 312, 312)],
                        out_hbm.at[c, pl.ds(base + j * 312, 312)])

    @pl.when(s == NS - 1)
    def _():
        pltpu.sync_copy(table.at[pl.ds(N - 16, 16)],
                        out_hbm.at[c, pl.ds(N - 16, 16)])


@functools.partial(
    pl.kernel,
    out_type=jax.ShapeDtypeStruct((NC, N, D), jnp.float32),
    mesh=_sc_mesh(),
    scratch_types=[
        pltpu.VMEM((CH,), jnp.int32),
        pltpu.VMEM((CH,), jnp.int32),
        pltpu.VMEM((TAIL,), jnp.int32),
        pltpu.VMEM((TAIL,), jnp.int32),
        pltpu.VMEM((CH, D), jnp.float32),
        pltpu.VMEM((CH, D), jnp.float32),
        pltpu.VMEM((ZCH, D), jnp.float32),
        pltpu.VMEM_SHARED((N, D), jnp.float32),
        pltpu.SemaphoreType.DMA,
        pltpu.SemaphoreType.DMA,
    ],
)
def _deg_sc_kernel(srci_hbm, dsti_hbm, h_out,
                   sbuf, dbuf, sbuf_t, dbuf_t, e0, e1, zbuf, htab,
                   semA, semB):
    c = lax.axis_index("c")
    s = lax.axis_index("s")
    wid = s * NC + c
    ones = jnp.ones((16,), jnp.float32)
    zeros = jnp.zeros((16,), jnp.float32)

    def fill(i, carry):
        r = i // 8
        col = (i % 8) * 16
        e0[r, pl.ds(col, 16)] = zeros
        e1[r, pl.ds(col, 16)] = zeros
        return carry

    lax.fori_loop(0, CH * 8, fill, 0)

    def mark(i, carry):
        e0[i, pl.ds(0, 16)] = ones
        e1[i, pl.ds(64, 16)] = ones
        return carry

    lax.fori_loop(0, CH, mark, 0)
    _zero_my_slice(zbuf, htab, s)
    plsc.subcore_barrier()

    base = wid * EPT

    def chunk(k, carry):
        off = base + k * CH
        pltpu.sync_copy(srci_hbm.at[pl.ds(off, CH)], sbuf)
        pltpu.sync_copy(dsti_hbm.at[pl.ds(off, CH)], dbuf)
        pltpu.async_copy(e0, htab.at[sbuf], semA, add=True)
        pltpu.async_copy(e1, htab.at[dbuf], semB, add=True)
        pltpu.make_async_copy(e0, htab.at[sbuf], semA).wait()
        pltpu.make_async_copy(e1, htab.at[dbuf], semB).wait()
        return carry

    lax.fori_loop(0, NFULL, chunk, 0)

    offt = base + NFULL * CH
    pltpu.sync_copy(srci_hbm.at[pl.ds(offt, TAIL)], sbuf_t)
    pltpu.sync_copy(dsti_hbm.at[pl.ds(offt, TAIL)], dbuf_t)
    pltpu.sync_copy(e0.at[pl.ds(0, TAIL)], htab.at[sbuf_t], add=True)
    pltpu.sync_copy(e1.at[pl.ds(0, TAIL)], htab.at[dbuf_t], add=True)

    plsc.subcore_barrier()
    _write_back(htab, h_out, c, s)


def _sup_body(h0_ref, h1_ref, x_ref, w_ref, b_ref,
              sup_ref, rs_dr_ref, dc_ref):
    dr = h0_ref[:, 0:1] + h1_ref[:, 0:1]
    dc = h0_ref[:, 64:65] + h1_ref[:, 64:65]
    rs_dc = lax.rsqrt(jnp.maximum(dc, 1.0))
    rs_dr_ref[...] = lax.rsqrt(jnp.maximum(dr, 1.0))
    dc_ref[...] = dc
    acc = jnp.dot(x_ref[...], w_ref[...], preferred_element_type=jnp.float32)
    sup_ref[...] = (acc + b_ref[...]) * rs_dc


_sup_call = pl.pallas_call(
    _sup_body,
    out_shape=(
        jax.ShapeDtypeStruct((N, D), jnp.float32),
        jax.ShapeDtypeStruct((N, 1), jnp.float32),
        jax.ShapeDtypeStruct((N, 1), jnp.float32),
    ),
)


@functools.partial(
    pl.kernel,
    out_type=jax.ShapeDtypeStruct((NC, N, D), jnp.float32),
    mesh=_sc_mesh(),
    scratch_types=[
        pltpu.VMEM((CH,), jnp.int32),
        pltpu.VMEM((CH,), jnp.int32),
        pltpu.VMEM((CH, D), jnp.float32),
        pltpu.VMEM((CH,), jnp.int32),
        pltpu.VMEM((CH,), jnp.int32),
        pltpu.VMEM((CH, D), jnp.float32),
        pltpu.VMEM((TAIL,), jnp.int32),
        pltpu.VMEM((TAIL,), jnp.int32),
        pltpu.VMEM((TAIL, D), jnp.float32),
        pltpu.VMEM((ZCH, D), jnp.float32),
        pltpu.VMEM_SHARED((N, D), jnp.float32),
        pltpu.SemaphoreType.DMA,
        pltpu.SemaphoreType.DMA,
    ],
)
def _spmm_kernel(tab_hbm, dsti_hbm, srci_hbm, z_out,
                 dbuf0, sbuf0, rows0, dbuf1, sbuf1, rows1,
                 dbuf_t, sbuf_t, rows_t, zbuf, zacc, sem0, sem1):
    c = lax.axis_index("c")
    s = lax.axis_index("s")
    wid = s * NC + c
    _zero_my_slice(zbuf, zacc, s)
    plsc.subcore_barrier()

    base = wid * EPT

    def load_idx(k, dbuf, sbuf):
        off = base + k * CH
        pltpu.sync_copy(dsti_hbm.at[pl.ds(off, CH)], dbuf)
        pltpu.sync_copy(srci_hbm.at[pl.ds(off, CH)], sbuf)

    load_idx(0, dbuf0, sbuf0)
    pltpu.async_copy(tab_hbm.at[dbuf0], rows0, sem0)

    def body(i, carry):
        load_idx(2 * i + 1, dbuf1, sbuf1)
        pltpu.async_copy(tab_hbm.at[dbuf1], rows1, sem1)
        pltpu.make_async_copy(tab_hbm.at[dbuf0], rows0, sem0).wait()
        pltpu.sync_copy(rows0, zacc.at[sbuf0], add=True)

        @pl.when(i < NFULL // 2 - 1)
        def _():
            load_idx(2 * i + 2, dbuf0, sbuf0)
            pltpu.async_copy(tab_hbm.at[dbuf0], rows0, sem0)

        pltpu.make_async_copy(tab_hbm.at[dbuf1], rows1, sem1).wait()
        pltpu.sync_copy(rows1, zacc.at[sbuf1], add=True)
        return carry

    lax.fori_loop(0, NFULL // 2, body, 0)

    offt = base + NFULL * CH
    pltpu.sync_copy(dsti_hbm.at[pl.ds(offt, TAIL)], dbuf_t)
    pltpu.sync_copy(srci_hbm.at[pl.ds(offt, TAIL)], sbuf_t)
    pltpu.async_copy(tab_hbm.at[dbuf_t], rows_t, sem0).wait()
    pltpu.sync_copy(rows_t, zacc.at[sbuf_t], add=True)

    plsc.subcore_barrier()
    _write_back(zacc, z_out, c, s)


def _assign_body(z0_ref, z1_ref, rs_ref, wt_ref, bt_ref, dc_ref,
                 apad_ref, cs_ref, nl_ref):
    z = (z0_ref[...] + z1_ref[...]) * rs_ref[...]
    g = SELU_SCALE * jnp.where(z > 0, z, SELU_ALPHA * (jnp.exp(z) - 1.0))
    logits = jnp.dot(g, wt_ref[...], preferred_element_type=jnp.float32) + bt_ref[...]
    m = jnp.max(logits, axis=1, keepdims=True)
    e = jnp.exp(logits - m)
    a = e / jnp.sum(e, axis=1, keepdims=True)
    apad_ref[...] = jnp.concatenate(
        [a, jnp.zeros((N, D - K), jnp.float32)], axis=1
    )
    cs_ref[...] = jnp.sum(a, axis=0, keepdims=True)
    nl_ref[...] = jnp.sum(dc_ref[...] * a, axis=0, keepdims=True)


_assign_call = pl.pallas_call(
    _assign_body,
    out_shape=(
        jax.ShapeDtypeStruct((N, D), jnp.float32),
        jax.ShapeDtypeStruct((1, K), jnp.float32),
        jax.ShapeDtypeStruct((1, K), jnp.float32),
    ),
)


def _loss_body(as0_ref, as1_ref, apad_ref, cs_ref, nl_ref, loss_ref):
    tp = jnp.sum((as0_ref[...] + as1_ref[...]) * apad_ref[...])
    nl = nl_ref[...]
    cs = cs_ref[...]
    e_f = jnp.float32(E)
    tn = jnp.sum(nl * nl) / 2.0 / e_f
    spectral = -(tp - tn) / 2.0 / e_f
    cluster = jnp.sqrt(jnp.sum(cs * cs)) / jnp.float32(N) * np.sqrt(float(K)) - 1.0
    loss_ref[...] = jnp.full((1, 1), spectral + cluster, jnp.float32)


_loss_call = pl.pallas_call(
    _loss_body,
    out_shape=jax.ShapeDtypeStruct((1, 1), jnp.float32),
)


def kernel(edge_index, features, W_gcn, b_gcn, W_t, b_t):
    src = edge_index[0].astype(jnp.int32)
    dst = edge_index[1].astype(jnp.int32)

    h = _deg_sc_kernel(src, dst)
    support2, rs_dr, dc_col = _sup_call(
        h[0], h[1], features, W_gcn, b_gcn.reshape(1, D)
    )
    zp = _spmm_kernel(support2, dst, src)
    apad, cs, nl = _assign_call(
        zp[0], zp[1], rs_dr, W_t, b_t.reshape(1, K), dc_col
    )
    asp = _spmm_kernel(apad, dst, src)
    loss = _loss_call(asp[0], asp[1], apad, cs, nl)
    return loss[0, 0]


# speedup vs baseline: 16.4680x; 1.1255x over previous
import functools

import jax
import jax.numpy as jnp
import numpy as np
from jax import lax
from jax.experimental import pallas as pl
from jax.experimental.pallas import tpu as pltpu
from jax.experimental.pallas import tpu_sc as plsc

N = 10000
E = 320000
D = 128
K = 16

NC = 2
NS = 16
NW = NC * NS
EPT = E // NW
CH = 128
NFULL = EPT // CH
TAIL = EPT - NFULL * CH
WB_A = 624
WB_OFF = WB_A * (NS - 1)
WB_B = N - WB_OFF
ZCH = 104

SELU_ALPHA = 1.6732632423543772
SELU_SCALE = 1.0507009873554805


def _sc_mesh():
    return plsc.VectorSubcoreMesh(
        core_axis_name="c", subcore_axis_name="s", num_cores=NC, num_subcores=NS
    )


def _zero_my_slice(zbuf, table, s):
    zeros = jnp.zeros((16,), jnp.float32)

    def zb(i, carry):
        zbuf[i // 8, pl.ds((i % 8) * 16, 16)] = zeros
        return carry

    lax.fori_loop(0, ZCH * 8, zb, 0)
    base = s * WB_A
    for j in range(6):
        pltpu.sync_copy(zbuf, table.at[pl.ds(base + j * ZCH, ZCH)])

    @pl.when(s == NS - 1)
    def _():
        pltpu.sync_copy(zbuf.at[pl.ds(0, 16)], table.at[pl.ds(N - 16, 16)])


def _write_back(table, out_hbm, c, s):
    base = s * WB_A
    for j in range(2):
        pltpu.sync_copy(table.at[pl.ds(base + j * 312, 312)],
                        out_hbm.at[c, pl.ds(base + j * 312, 312)])

    @pl.when(s == NS - 1)
    def _():
        pltpu.sync_copy(table.at[pl.ds(N - 16, 16)],
                        out_hbm.at[c, pl.ds(N - 16, 16)])


@functools.partial(
    pl.kernel,
    out_type=jax.ShapeDtypeStruct((NC, N, D), jnp.float32),
    mesh=_sc_mesh(),
    scratch_types=[
        pltpu.VMEM((CH,), jnp.int32),
        pltpu.VMEM((CH,), jnp.int32),
        pltpu.VMEM((CH,), jnp.int32),
        pltpu.VMEM((CH,), jnp.int32),
        pltpu.VMEM((TAIL,), jnp.int32),
        pltpu.VMEM((TAIL,), jnp.int32),
        pltpu.VMEM((CH, D), jnp.float32),
        pltpu.VMEM((CH, D), jnp.float32),
        pltpu.VMEM((ZCH, D), jnp.float32),
        pltpu.VMEM_SHARED((N, D), jnp.float32),
        pltpu.SemaphoreType.DMA,
        pltpu.SemaphoreType.DMA,
        pltpu.SemaphoreType.DMA,
        pltpu.SemaphoreType.DMA,
    ],
)
def _deg_sc_kernel(srci_hbm, dsti_hbm, h_out,
                   sbuf0, dbuf0, sbuf1, dbuf1, sbuf_t, dbuf_t, e0, e1,
                   zbuf, htab, semA, semB, semC, semD):
    c = lax.axis_index("c")
    s = lax.axis_index("s")
    wid = s * NC + c
    ones = jnp.ones((16,), jnp.float32)
    zeros = jnp.zeros((16,), jnp.float32)

    def fill(i, carry):
        r = i // 8
        col = (i % 8) * 16
        e0[r, pl.ds(col, 16)] = zeros
        e1[r, pl.ds(col, 16)] = zeros
        return carry

    lax.fori_loop(0, CH * 8, fill, 0)

    def mark(i, carry):
        e0[i, pl.ds(0, 16)] = ones
        e1[i, pl.ds(64, 16)] = ones
        return carry

    lax.fori_loop(0, CH, mark, 0)
    _zero_my_slice(zbuf, htab, s)
    plsc.subcore_barrier()

    base = wid * EPT

    def load_idx(k, sb, db):
        off = base + k * CH
        pltpu.sync_copy(srci_hbm.at[pl.ds(off, CH)], sb)
        pltpu.sync_copy(dsti_hbm.at[pl.ds(off, CH)], db)

    load_idx(0, sbuf0, dbuf0)
    pltpu.async_copy(e0, htab.at[sbuf0], semA, add=True)
    pltpu.async_copy(e1, htab.at[dbuf0], semB, add=True)

    def chunk(i, carry):
        load_idx(2 * i + 1, sbuf1, dbuf1)
        pltpu.async_copy(e0, htab.at[sbuf1], semC, add=True)
        pltpu.async_copy(e1, htab.at[dbuf1], semD, add=True)
        pltpu.make_async_copy(e0, htab.at[sbuf0], semA).wait()
        pltpu.make_async_copy(e1, htab.at[dbuf0], semB).wait()

        @pl.when(i < NFULL // 2 - 1)
        def _():
            load_idx(2 * i + 2, sbuf0, dbuf0)
            pltpu.async_copy(e0, htab.at[sbuf0], semA, add=True)
            pltpu.async_copy(e1, htab.at[dbuf0], semB, add=True)

        pltpu.make_async_copy(e0, htab.at[sbuf1], semC).wait()
        pltpu.make_async_copy(e1, htab.at[dbuf1], semD).wait()
        return carry

    lax.fori_loop(0, NFULL // 2, chunk, 0)

    offt = base + NFULL * CH
    pltpu.sync_copy(srci_hbm.at[pl.ds(offt, TAIL)], sbuf_t)
    pltpu.sync_copy(dsti_hbm.at[pl.ds(offt, TAIL)], dbuf_t)
    pltpu.sync_copy(e0.at[pl.ds(0, TAIL)], htab.at[sbuf_t], add=True)
    pltpu.sync_copy(e1.at[pl.ds(0, TAIL)], htab.at[dbuf_t], add=True)

    plsc.subcore_barrier()
    _write_back(htab, h_out, c, s)


def _sup_body(h0_ref, h1_ref, x_ref, w_ref, b_ref,
              sup_ref, rs_dr_ref, dc_ref):
    dr = h0_ref[:, 0:1] + h1_ref[:, 0:1]
    dc = h0_ref[:, 64:65] + h1_ref[:, 64:65]
    rs_dc = lax.rsqrt(jnp.maximum(dc, 1.0))
    rs_dr_ref[...] = lax.rsqrt(jnp.maximum(dr, 1.0))
    dc_ref[...] = dc
    acc = jnp.dot(x_ref[...], w_ref[...], preferred_element_type=jnp.float32)
    sup_ref[...] = (acc + b_ref[...]) * rs_dc


_sup_call = pl.pallas_call(
    _sup_body,
    out_shape=(
        jax.ShapeDtypeStruct((N, D), jnp.float32),
        jax.ShapeDtypeStruct((N, 1), jnp.float32),
        jax.ShapeDtypeStruct((N, 1), jnp.float32),
    ),
)


@functools.partial(
    pl.kernel,
    out_type=jax.ShapeDtypeStruct((NC, N, D), jnp.float32),
    mesh=_sc_mesh(),
    scratch_types=[
        pltpu.VMEM((CH,), jnp.int32),
        pltpu.VMEM((CH,), jnp.int32),
        pltpu.VMEM((CH, D), jnp.float32),
        pltpu.VMEM((CH,), jnp.int32),
        pltpu.VMEM((CH,), jnp.int32),
        pltpu.VMEM((CH, D), jnp.float32),
        pltpu.VMEM((TAIL,), jnp.int32),
        pltpu.VMEM((TAIL,), jnp.int32),
        pltpu.VMEM((TAIL, D), jnp.float32),
        pltpu.VMEM((ZCH, D), jnp.float32),
        pltpu.VMEM_SHARED((N, D), jnp.float32),
        pltpu.SemaphoreType.DMA,
        pltpu.SemaphoreType.DMA,
    ],
)
def _spmm_kernel(tab_hbm, dsti_hbm, srci_hbm, z_out,
                 dbuf0, sbuf0, rows0, dbuf1, sbuf1, rows1,
                 dbuf_t, sbuf_t, rows_t, zbuf, zacc, sem0, sem1):
    c = lax.axis_index("c")
    s = lax.axis_index("s")
    wid = s * NC + c
    _zero_my_slice(zbuf, zacc, s)
    plsc.subcore_barrier()

    base = wid * EPT

    def load_idx(k, dbuf, sbuf):
        off = base + k * CH
        pltpu.sync_copy(dsti_hbm.at[pl.ds(off, CH)], dbuf)
        pltpu.sync_copy(srci_hbm.at[pl.ds(off, CH)], sbuf)

    load_idx(0, dbuf0, sbuf0)
    pltpu.async_copy(tab_hbm.at[dbuf0], rows0, sem0)

    def body(i, carry):
        load_idx(2 * i + 1, dbuf1, sbuf1)
        pltpu.async_copy(tab_hbm.at[dbuf1], rows1, sem1)
        pltpu.make_async_copy(tab_hbm.at[dbuf0], rows0, sem0).wait()
        pltpu.sync_copy(rows0, zacc.at[sbuf0], add=True)

        @pl.when(i < NFULL // 2 - 1)
        def _():
            load_idx(2 * i + 2, dbuf0, sbuf0)
            pltpu.async_copy(tab_hbm.at[dbuf0], rows0, sem0)

        pltpu.make_async_copy(tab_hbm.at[dbuf1], rows1, sem1).wait()
        pltpu.sync_copy(rows1, zacc.at[sbuf1], add=True)
        return carry

    lax.fori_loop(0, NFULL // 2, body, 0)

    offt = base + NFULL * CH
    pltpu.sync_copy(dsti_hbm.at[pl.ds(offt, TAIL)], dbuf_t)
    pltpu.sync_copy(srci_hbm.at[pl.ds(offt, TAIL)], sbuf_t)
    pltpu.async_copy(tab_hbm.at[dbuf_t], rows_t, sem0).wait()
    pltpu.sync_copy(rows_t, zacc.at[sbuf_t], add=True)

    plsc.subcore_barrier()
    _write_back(zacc, z_out, c, s)


def _assign_body(z0_ref, z1_ref, rs_ref, wt_ref, bt_ref, dc_ref,
                 apad_ref, cs_ref, nl_ref):
    z = (z0_ref[...] + z1_ref[...]) * rs_ref[...]
    g = SELU_SCALE * jnp.where(z > 0, z, SELU_ALPHA * (jnp.exp(z) - 1.0))
    logits = jnp.dot(g, wt_ref[...], preferred_element_type=jnp.float32) + bt_ref[...]
    m = jnp.max(logits, axis=1, keepdims=True)
    e = jnp.exp(logits - m)
    a = e / jnp.sum(e, axis=1, keepdims=True)
    apad_ref[...] = jnp.concatenate(
        [a, jnp.zeros((N, D - K), jnp.float32)], axis=1
    )
    cs_ref[...] = jnp.sum(a, axis=0, keepdims=True)
    nl_ref[...] = jnp.sum(dc_ref[...] * a, axis=0, keepdims=True)


_assign_call = pl.pallas_call(
    _assign_body,
    out_shape=(
        jax.ShapeDtypeStruct((N, D), jnp.float32),
        jax.ShapeDtypeStruct((1, K), jnp.float32),
        jax.ShapeDtypeStruct((1, K), jnp.float32),
    ),
)


def _loss_body(as0_ref, as1_ref, apad_ref, cs_ref, nl_ref, loss_ref):
    tp = jnp.sum((as0_ref[...] + as1_ref[...]) * apad_ref[...])
    nl = nl_ref[...]
    cs = cs_ref[...]
    e_f = jnp.float32(E)
    tn = jnp.sum(nl * nl) / 2.0 / e_f
    spectral = -(tp - tn) / 2.0 / e_f
    cluster = jnp.sqrt(jnp.sum(cs * cs)) / jnp.float32(N) * np.sqrt(float(K)) - 1.0
    loss_ref[...] = jnp.full((1, 1), spectral + cluster, jnp.float32)


_loss_call = pl.pallas_call(
    _loss_body,
    out_shape=jax.ShapeDtypeStruct((1, 1), jnp.float32),
)


def kernel(edge_index, features, W_gcn, b_gcn, W_t, b_t):
    src = edge_index[0].astype(jnp.int32)
    dst = edge_index[1].astype(jnp.int32)

    h = _deg_sc_kernel(src, dst)
    support2, rs_dr, dc_col = _sup_call(
        h[0], h[1], features, W_gcn, b_gcn.reshape(1, D)
    )
    zp = _spmm_kernel(support2, dst, src)
    apad, cs, nl = _assign_call(
        zp[0], zp[1], rs_dr, W_t, b_t.reshape(1, K), dc_col
    )
    asp = _spmm_kernel(apad, dst, src)
    loss = _loss_call(asp[0], asp[1], apad, cs, nl)
    return loss[0, 0]
